# baseline (speedup 1.0000x reference)
import functools

import jax
import jax.numpy as jnp
from jax import lax
from jax.experimental import pallas as pl
from jax.experimental.pallas import tpu as pltpu
from jax.experimental.pallas import tpu_sc as plsc

_N = 10000
_E = 320000
_D = 128
_G = 128
_MF = 16
_DH = 128

_NP = 10240
_EP = 327680
_CHUNK = 128
_NCHUNKS = _EP // _CHUNK
_NC, _NS = 2, 16
_CPT = _NCHUNKS // (_NC * _NS)
_CPT_A = 160
_CPT_B = 0
_ROWS_PT = _NP // _NS
_NB = _NP // 256

_SELU_ALPHA = 1.6732632423543772
_SELU_SCALE = 1.0507009873554805

_P = lax.Precision.HIGHEST
_MESH = plsc.VectorSubcoreMesh(core_axis_name="c", subcore_axis_name="s")


def _selu(z):
    return _SELU_SCALE * jnp.where(z > 0, z, _SELU_ALPHA * (jnp.exp(z) - 1.0))


def _dinv(d0, d1):
    return lax.rsqrt(1.0 + d0[:, 0:1] + d1[:, 0:1])


def _fill_rows(buf, nrows, width, value):
    vec = jnp.full((16,), value, jnp.float32)

    def body(r, carry):
        for k in range(width // 16):
            buf[r, pl.ds(k * 16, 16)] = vec
        return carry

    lax.fori_loop(0, nrows, body, 0)


def _deg_body(dst_hbm, out_hbm, db0, db1, rbuf, hist, dsem0, dsem1):
    c = lax.axis_index("c")
    s = lax.axis_index("s")
    _fill_rows(rbuf, _CHUNK, _D, 0.0)
    row0 = s * _ROWS_PT
    for k in range(_ROWS_PT // _CHUNK):
        pltpu.sync_copy(rbuf, hist.at[pl.ds(row0 + k * _CHUNK, _CHUNK)])
    plsc.subcore_barrier()
    _fill_rows(rbuf, _CHUNK, _D, 1.0)
    ebase = (c * _NS + s) * _CPT * _CHUNK
    dbufs = (db0, db1)
    dsems = (dsem0, dsem1)
    pltpu.async_copy(dst_hbm.at[pl.ds(ebase, _CHUNK)], db0, dsem0)

    def body(t, carry):
        for par in range(2):
            j = 2 * t + par
            np_ = 1 - par

            @pl.when(j + 1 < _CPT)
            def _():
                pltpu.async_copy(
                    dst_hbm.at[pl.ds(ebase + (j + 1) * _CHUNK, _CHUNK)],
                    dbufs[np_], dsems[np_])

            pltpu.make_async_copy(
                dst_hbm.at[pl.ds(ebase + j * _CHUNK, _CHUNK)], dbufs[par],
                dsems[par]).wait()
            pltpu.sync_copy(rbuf, hist.at[dbufs[par]], add=True)
        return carry

    lax.fori_loop(0, _CPT // 2, body, 0)
    plsc.subcore_barrier()
    pltpu.sync_copy(
        hist.at[pl.ds(row0, _ROWS_PT)],
        out_hbm.at[pl.ds(c * _NP + row0, _ROWS_PT)],
    )


def _sc_degree(dst_flat):
    return pl.kernel(
        _deg_body,
        out_type=jax.ShapeDtypeStruct((2 * _NP, _D), jnp.float32),
        mesh=_MESH,
        scratch_types=[
            pltpu.VMEM((_CHUNK,), jnp.int32),
            pltpu.VMEM((_CHUNK,), jnp.int32),
            pltpu.VMEM((_CHUNK, _D), jnp.float32),
            pltpu.VMEM_SHARED((_NP, _D), jnp.float32),
            pltpu.SemaphoreType.DMA,
            pltpu.SemaphoreType.DMA,
        ],
    )(dst_flat)


def _edge_body(hs_hbm, src_hbm, dst_hbm, out_hbm, sb0, sb1, db0, db1,
               gb0, gb1, acc, gsem0, gsem1, ssem0, ssem1, dsem0, dsem1):
    c = lax.axis_index("c")
    s = lax.axis_index("s")
    cpt = jnp.where(c == 0, _CPT_A, _CPT_B)
    cbase = s * _CPT_A
    row0 = s * _ROWS_PT

    @pl.when(c == 0)
    def _():
        _fill_rows(gb0, _CHUNK, _D, 0.0)
        for k in range(_ROWS_PT // _CHUNK):
            pltpu.sync_copy(gb0, acc.at[pl.ds(row0 + k * _CHUNK, _CHUNK)])
    plsc.subcore_barrier()

    gbufs = (gb0, gb1)
    gsems = (gsem0, gsem1)
    sbufs = (sb0, sb1)
    ssems = (ssem0, ssem1)
    dbufs = (db0, db1)
    dsems = (dsem0, dsem1)

    ebase = cbase * _CHUNK

    @pl.when(cpt > 0)
    def _():
        pltpu.sync_copy(src_hbm.at[pl.ds(ebase, _CHUNK)], sb0)
        pltpu.async_copy(hs_hbm.at[sb0], gb0, gsem0)
        pltpu.async_copy(
            src_hbm.at[pl.ds(ebase + _CHUNK, _CHUNK)], sb1, ssem1)
        pltpu.async_copy(dst_hbm.at[pl.ds(ebase, _CHUNK)], db0, dsem0)

    def body(t, carry):
        for par in range(2):
            j = 2 * t + par
            np_ = 1 - par

            pltpu.make_async_copy(
                hs_hbm.at[sbufs[par]], gbufs[par], gsems[par]).wait()

            @pl.when(j + 2 < cpt)
            def _():
                pltpu.async_copy(
                    src_hbm.at[pl.ds(ebase + (j + 2) * _CHUNK, _CHUNK)],
                    sbufs[par], ssems[par])

            @pl.when(j + 1 < cpt)
            def _():
                pltpu.make_async_copy(
                    src_hbm.at[pl.ds(ebase + (j + 1) * _CHUNK, _CHUNK)],
                    sbufs[np_], ssems[np_]).wait()
                pltpu.async_copy(
                    hs_hbm.at[sbufs[np_]], gbufs[np_], gsems[np_])
                pltpu.async_copy(
                    dst_hbm.at[pl.ds(ebase + (j + 1) * _CHUNK, _CHUNK)],
                    dbufs[np_], dsems[np_])

            pltpu.make_async_copy(
                dst_hbm.at[pl.ds(ebase + j * _CHUNK, _CHUNK)], dbufs[par],
                dsems[par]).wait()
            pltpu.sync_copy(gbufs[par], acc.at[dbufs[par]], add=True)
        return carry

    lax.fori_loop(0, cpt // 2, body, 0)
    plsc.subcore_barrier()

    @pl.when(c == 0)
    def _():
        pltpu.sync_copy(
            acc.at[pl.ds(row0, _ROWS_PT)],
            out_hbm.at[pl.ds(row0, _ROWS_PT)],
        )


def _sc_edge_scatter(hs, src2d, dst2d):
    return pl.kernel(
        _edge_body,
        out_type=jax.ShapeDtypeStruct((_NP, _D), jnp.float32),
        mesh=_MESH,
        scratch_types=[
            pltpu.VMEM((_CHUNK,), jnp.int32),
            pltpu.VMEM((_CHUNK,), jnp.int32),
            pltpu.VMEM((_CHUNK,), jnp.int32),
            pltpu.VMEM((_CHUNK,), jnp.int32),
            pltpu.VMEM((_CHUNK, _D), jnp.float32),
            pltpu.VMEM((_CHUNK, _D), jnp.float32),
            pltpu.VMEM_SHARED((_NP, _D), jnp.float32),
            pltpu.SemaphoreType.DMA,
            pltpu.SemaphoreType.DMA,
            pltpu.SemaphoreType.DMA,
            pltpu.SemaphoreType.DMA,
            pltpu.SemaphoreType.DMA,
            pltpu.SemaphoreType.DMA,
        ],
    )(hs, src2d, dst2d)


def _pre_body(x_ref, w_ref, d0_ref, d1_ref, o_ref):
    dinv = _dinv(d0_ref[...], d1_ref[...])
    h = jnp.dot(x_ref[...], w_ref[...], precision=_P,
                preferred_element_type=jnp.float32)
    o_ref[...] = h * dinv


def _tc_pre(x_pad, w, degp):
    return pl.pallas_call(
        _pre_body,
        grid=(_NB,),
        in_specs=[
            pl.BlockSpec((256, _D), lambda i: (i, 0)),
            pl.BlockSpec((_D, _DH), lambda i: (0, 0)),
            pl.BlockSpec((256, _D), lambda i: (i, 0)),
            pl.BlockSpec((256, _D), lambda i: (_NB + i, 0)),
        ],
        out_specs=pl.BlockSpec((256, _DH), lambda i: (i, 0)),
        out_shape=jax.ShapeDtypeStruct((_NP, _DH), jnp.float32),
    )(x_pad, w, degp, degp)


def _postA_body(p0, hs, d0, d1, b, a_ref, s_ref, acc):
    i = pl.program_id(0)
    dinv = _dinv(d0[...], d1[...])
    z = dinv * (p0[...] + hs[...]) + b[0:1, :]
    a = _selu(z)
    a_ref[...] = a
    rows = i * 256 + lax.broadcasted_iota(jnp.int32, (256, 1), 0)
    am = jnp.where(rows < _N, a, 0.0)

    @pl.when(i == 0)
    def _():
        acc[...] = jnp.zeros_like(acc)

    acc[0:1, :] += jnp.sum(am, axis=0, keepdims=True)
    acc[1:2, :] += jnp.sum(am * am, axis=0, keepdims=True)

    @pl.when(i == _NB - 1)
    def _():
        s_ref[...] = acc[...]


def _tc_postA(parts, hs, degp, b):
    return pl.pallas_call(
        _postA_body,
        grid=(_NB,),
        in_specs=[
            pl.BlockSpec((256, _DH), lambda i: (i, 0)),
            pl.BlockSpec((256, _DH), lambda i: (i, 0)),
            pl.BlockSpec((256, _D), lambda i: (i, 0)),
            pl.BlockSpec((256, _D), lambda i: (_NB + i, 0)),
            pl.BlockSpec((1, _DH), lambda i: (0, 0)),
        ],
        out_specs=[
            pl.BlockSpec((256, _DH), lambda i: (i, 0)),
            pl.BlockSpec((8, _DH), lambda i: (0, 0)),
        ],
        out_shape=[
            jax.ShapeDtypeStruct((_NP, _DH), jnp.float32),
            jax.ShapeDtypeStruct((8, _DH), jnp.float32),
        ],
        scratch_shapes=[pltpu.VMEM((8, _DH), jnp.float32)],
    )(parts, hs, degp, degp, b)


def _postB_body(a, s, g, be, w, d0, d1, o_ref):
    mu = s[0:1, :] * (1.0 / _N)
    var = s[1:2, :] * (1.0 / _N) - mu * mu
    rstd = lax.rsqrt(var + 1e-5)
    h = (a[...] - mu) * rstd * g[0:1, :] + be[0:1, :]
    dinv = _dinv(d0[...], d1[...])
    o_ref[...] = jnp.dot(h, w[...], precision=_P,
                         preferred_element_type=jnp.float32) * dinv


def _tc_postB(a, sums, g, be, w, degp):
    return pl.pallas_call(
        _postB_body,
        grid=(_NB,),
        in_specs=[
            pl.BlockSpec((256, _DH), lambda i: (i, 0)),
            pl.BlockSpec((8, _DH), lambda i: (0, 0)),
            pl.BlockSpec((1, _DH), lambda i: (0, 0)),
            pl.BlockSpec((1, _DH), lambda i: (0, 0)),
            pl.BlockSpec((_DH, _DH), lambda i: (0, 0)),
            pl.BlockSpec((256, _D), lambda i: (i, 0)),
            pl.BlockSpec((256, _D), lambda i: (_NB + i, 0)),
        ],
        out_specs=pl.BlockSpec((256, _DH), lambda i: (i, 0)),
        out_shape=jax.ShapeDtypeStruct((_NP, _DH), jnp.float32),
    )(a, sums, g, be, w, degp, degp)


def _final_body(p0, hs, d0, d1, b, bt, mol, wa, wb, bf1, wf2, bf2,
                o_ref, hg_acc):
    i = pl.program_id(0)
    dinv = _dinv(d0[...], d1[...])
    z = dinv * (p0[...] + hs[...]) + b[0:1, :]
    a = _selu(z)
    oh = (bt[...] == lax.broadcasted_iota(jnp.int32, (256, _G), 1)
          ).astype(jnp.float32)
    part = lax.dot_general(oh, a, (((0,), (0,)), ((), ())), precision=_P,
                           preferred_element_type=jnp.float32)

    @pl.when(i == 0)
    def _():
        hg_acc[...] = jnp.zeros_like(hg_acc)

    hg_acc[...] += part

    @pl.when(i == _NB - 1)
    def _():
        hg = hg_acc[...]
        h = (jnp.dot(hg, wa[...], precision=_P,
                     preferred_element_type=jnp.float32)
             + jnp.dot(mol[...], wb[...], precision=_P,
                       preferred_element_type=jnp.float32)
             + bf1[0:1, :])
        h = _selu(h)
        res = jnp.dot(h, wf2[...], precision=_P,
                      preferred_element_type=jnp.float32)
        o_ref[...] = res[:, 0:1] + bf2[0, 0]


def _tc_final(parts, hs, degp, b, batch2d, mol, wa, wb, bf1, wf2p, bf2):
    return pl.pallas_call(
        _final_body,
        grid=(_NB,),
        in_specs=[
            pl.BlockSpec((256, _DH), lambda i: (i, 0)),
            pl.BlockSpec((256, _DH), lambda i: (i, 0)),
            pl.BlockSpec((256, _D), lambda i: (i, 0)),
            pl.BlockSpec((256, _D), lambda i: (_NB + i, 0)),
            pl.BlockSpec((1, _DH), lambda i: (0, 0)),
            pl.BlockSpec((256, 1), lambda i: (i, 0)),
            pl.BlockSpec((_G, _MF), lambda i: (0, 0)),
            pl.BlockSpec((_DH, _DH), lambda i: (0, 0)),
            pl.BlockSpec((_MF, _DH), lambda i: (0, 0)),
            pl.BlockSpec((1, _DH), lambda i: (0, 0)),
            pl.BlockSpec((_DH, _DH), lambda i: (0, 0)),
            pl.BlockSpec((1, 1), lambda i: (0, 0)),
        ],
        out_specs=pl.BlockSpec((_G, 1), lambda i: (0, 0)),
        out_shape=jax.ShapeDtypeStruct((_G, 1), jnp.float32),
        scratch_shapes=[pltpu.VMEM((_G, _DH), jnp.float32)],
    )(parts, hs, degp, degp, b, batch2d, mol, wa, wb, bf1, wf2p, bf2)


def kernel(x, edge_index, batch, mol_feats, W1, b1, g1, be1, W2, b2, g2, be2,
           W3, b3, Wf1, bf1, Wf2, bf2):
    f32 = jnp.float32
    npad = _NP - _N
    epad = _EP - _E

    src = jnp.concatenate(
        [edge_index[0], jnp.full((epad,), _N, jnp.int32)])
    dst = jnp.concatenate(
        [edge_index[1], _N + (jnp.arange(epad, dtype=jnp.int32) % 128)])
    x_pad = jnp.concatenate([x, jnp.zeros((npad, _D), f32)], axis=0)
    batch2d = jnp.concatenate(
        [batch, jnp.full((npad,), _G, jnp.int32)]).reshape(_NP, 1)

    b1r = b1.reshape(1, _DH)
    b2r = b2.reshape(1, _DH)
    b3r = b3.reshape(1, _DH)
    g1r = g1.reshape(1, _DH)
    g2r = g2.reshape(1, _DH)
    be1r = be1.reshape(1, _DH)
    be2r = be2.reshape(1, _DH)
    bf1r = bf1.reshape(1, -1)
    wa = Wf1[:_DH]
    wb = Wf1[_DH:]
    wf2p = jnp.concatenate([Wf2, jnp.zeros((Wf2.shape[0], _DH - Wf2.shape[1]),
                                           f32)], axis=1)
    bf2r = bf2.reshape(1, 1)

    degp = _sc_degree(dst)

    hs1 = _tc_pre(x_pad, W1, degp)
    p1 = _sc_edge_scatter(hs1, src, dst)
    a1, s1 = _tc_postA(p1, hs1, degp, b1r)
    hs2 = _tc_postB(a1, s1, g1r, be1r, W2, degp)

    p2 = _sc_edge_scatter(hs2, src, dst)
    a2, s2 = _tc_postA(p2, hs2, degp, b2r)
    hs3 = _tc_postB(a2, s2, g2r, be2r, W3, degp)

    p3 = _sc_edge_scatter(hs3, src, dst)
    out = _tc_final(p3, hs3, degp, b3r, batch2d, mol_feats, wa, wb, bf1r,
                    wf2p, bf2r)
    return out


# speedup vs baseline: 1.3750x; 1.3750x over previous
import functools

import jax
import jax.numpy as jnp
from jax import lax
from jax.experimental import pallas as pl
from jax.experimental.pallas import tpu as pltpu
from jax.experimental.pallas import tpu_sc as plsc

_N = 10000
_E = 320000
_D = 128
_G = 128
_MF = 16
_DH = 128

_NP = 10240
_EP = 327680
_CHUNK = 128
_NCHUNKS = _EP // _CHUNK
_NC, _NS = 2, 16
_CPT = _NCHUNKS // (_NC * _NS)
_CPT_A = 140
_CPT_B = 20
_ROWS_PT = _NP // _NS
_NB = _NP // 256

_SELU_ALPHA = 1.6732632423543772
_SELU_SCALE = 1.0507009873554805

_P = lax.Precision.HIGHEST
_MESH = plsc.VectorSubcoreMesh(core_axis_name="c", subcore_axis_name="s")


def _selu(z):
    return _SELU_SCALE * jnp.where(z > 0, z, _SELU_ALPHA * (jnp.exp(z) - 1.0))


def _dinv(d0, d1):
    return lax.rsqrt(1.0 + d0[:, 0:1] + d1[:, 0:1])


def _fill_rows(buf, nrows, width, value):
    vec = jnp.full((16,), value, jnp.float32)

    def body(r, carry):
        for k in range(width // 16):
            buf[r, pl.ds(k * 16, 16)] = vec
        return carry

    lax.fori_loop(0, nrows, body, 0)


def _deg_body(dst_hbm, out_hbm, db0, db1, rbuf, hist, dsem0, dsem1):
    c = lax.axis_index("c")
    s = lax.axis_index("s")
    _fill_rows(rbuf, _CHUNK, _D, 0.0)
    row0 = s * _ROWS_PT
    for k in range(_ROWS_PT // _CHUNK):
        pltpu.sync_copy(rbuf, hist.at[pl.ds(row0 + k * _CHUNK, _CHUNK)])
    plsc.subcore_barrier()
    _fill_rows(rbuf, _CHUNK, _D, 1.0)
    ebase = (c * _NS + s) * _CPT * _CHUNK
    dbufs = (db0, db1)
    dsems = (dsem0, dsem1)
    pltpu.async_copy(dst_hbm.at[pl.ds(ebase, _CHUNK)], db0, dsem0)

    def body(t, carry):
        for par in range(2):
            j = 2 * t + par
            np_ = 1 - par

            @pl.when(j + 1 < _CPT)
            def _():
                pltpu.async_copy(
                    dst_hbm.at[pl.ds(ebase + (j + 1) * _CHUNK, _CHUNK)],
                    dbufs[np_], dsems[np_])

            pltpu.make_async_copy(
                dst_hbm.at[pl.ds(ebase + j * _CHUNK, _CHUNK)], dbufs[par],
                dsems[par]).wait()
            pltpu.sync_copy(rbuf, hist.at[dbufs[par]], add=True)
        return carry

    lax.fori_loop(0, _CPT // 2, body, 0)
    plsc.subcore_barrier()
    pltpu.sync_copy(
        hist.at[pl.ds(row0, _ROWS_PT)],
        out_hbm.at[pl.ds(c * _NP + row0, _ROWS_PT)],
    )


def _sc_degree(dst_flat):
    return pl.kernel(
        _deg_body,
        out_type=jax.ShapeDtypeStruct((2 * _NP, _D), jnp.float32),
        mesh=_MESH,
        scratch_types=[
            pltpu.VMEM((_CHUNK,), jnp.int32),
            pltpu.VMEM((_CHUNK,), jnp.int32),
            pltpu.VMEM((_CHUNK, _D), jnp.float32),
            pltpu.VMEM_SHARED((_NP, _D), jnp.float32),
            pltpu.SemaphoreType.DMA,
            pltpu.SemaphoreType.DMA,
        ],
    )(dst_flat)


def _edge_body(hs_hbm, src_hbm, dst_hbm, out_hbm, sb0, sb1, db0, db1,
               gb0, gb1, acc, gsem0, gsem1, ssem0, ssem1, dsem0, dsem1):
    c = lax.axis_index("c")
    s = lax.axis_index("s")
    cpt = jnp.where(c == 0, _CPT_A, _CPT_B)
    cbase = jnp.where(c == 0, s * _CPT_A, _NS * _CPT_A + s * _CPT_B)
    row0 = s * _ROWS_PT

    _fill_rows(gb0, _CHUNK, _D, 0.0)
    for k in range(_ROWS_PT // _CHUNK):
        pltpu.sync_copy(gb0, acc.at[pl.ds(row0 + k * _CHUNK, _CHUNK)])
    plsc.subcore_barrier()

    gbufs = (gb0, gb1)
    gsems = (gsem0, gsem1)
    sbufs = (sb0, sb1)
    ssems = (ssem0, ssem1)
    dbufs = (db0, db1)
    dsems = (dsem0, dsem1)

    ebase = cbase * _CHUNK

    @pl.when(cpt > 0)
    def _():
        pltpu.sync_copy(src_hbm.at[pl.ds(ebase, _CHUNK)], sb0)
        pltpu.async_copy(hs_hbm.at[sb0], gb0, gsem0)
        pltpu.async_copy(
            src_hbm.at[pl.ds(ebase + _CHUNK, _CHUNK)], sb1, ssem1)
        pltpu.async_copy(dst_hbm.at[pl.ds(ebase, _CHUNK)], db0, dsem0)

    def body(t, carry):
        for par in range(2):
            j = 2 * t + par
            np_ = 1 - par

            pltpu.make_async_copy(
                hs_hbm.at[sbufs[par]], gbufs[par], gsems[par]).wait()

            @pl.when(j + 2 < cpt)
            def _():
                pltpu.async_copy(
                    src_hbm.at[pl.ds(ebase + (j + 2) * _CHUNK, _CHUNK)],
                    sbufs[par], ssems[par])

            @pl.when(j + 1 < cpt)
            def _():
                pltpu.make_async_copy(
                    src_hbm.at[pl.ds(ebase + (j + 1) * _CHUNK, _CHUNK)],
                    sbufs[np_], ssems[np_]).wait()
                pltpu.async_copy(
                    hs_hbm.at[sbufs[np_]], gbufs[np_], gsems[np_])
                pltpu.async_copy(
                    dst_hbm.at[pl.ds(ebase + (j + 1) * _CHUNK, _CHUNK)],
                    dbufs[np_], dsems[np_])

            pltpu.make_async_copy(
                dst_hbm.at[pl.ds(ebase + j * _CHUNK, _CHUNK)], dbufs[par],
                dsems[par]).wait()
            pltpu.sync_copy(gbufs[par], acc.at[dbufs[par]], add=True)
        return carry

    lax.fori_loop(0, cpt // 2, body, 0)
    plsc.subcore_barrier()
    pltpu.sync_copy(
        acc.at[pl.ds(row0, _ROWS_PT)],
        out_hbm.at[pl.ds(c * _NP + row0, _ROWS_PT)],
    )


def _sc_edge_scatter(hs, src2d, dst2d):
    return pl.kernel(
        _edge_body,
        out_type=jax.ShapeDtypeStruct((2 * _NP, _D), jnp.float32),
        mesh=_MESH,
        scratch_types=[
            pltpu.VMEM((_CHUNK,), jnp.int32),
            pltpu.VMEM((_CHUNK,), jnp.int32),
            pltpu.VMEM((_CHUNK,), jnp.int32),
            pltpu.VMEM((_CHUNK,), jnp.int32),
            pltpu.VMEM((_CHUNK, _D), jnp.float32),
            pltpu.VMEM((_CHUNK, _D), jnp.float32),
            pltpu.VMEM_SHARED((_NP, _D), jnp.float32),
            pltpu.SemaphoreType.DMA,
            pltpu.SemaphoreType.DMA,
            pltpu.SemaphoreType.DMA,
            pltpu.SemaphoreType.DMA,
            pltpu.SemaphoreType.DMA,
            pltpu.SemaphoreType.DMA,
        ],
    )(hs, src2d, dst2d)


def _pre_body(x_ref, w_ref, d0_ref, d1_ref, o_ref):
    dinv = _dinv(d0_ref[...], d1_ref[...])
    h = jnp.dot(x_ref[...], w_ref[...], precision=_P,
                preferred_element_type=jnp.float32)
    o_ref[...] = h * dinv


def _tc_pre(x_pad, w, degp):
    return pl.pallas_call(
        _pre_body,
        grid=(_NB,),
        in_specs=[
            pl.BlockSpec((256, _D), lambda i: (i, 0)),
            pl.BlockSpec((_D, _DH), lambda i: (0, 0)),
            pl.BlockSpec((256, _D), lambda i: (i, 0)),
            pl.BlockSpec((256, _D), lambda i: (_NB + i, 0)),
        ],
        out_specs=pl.BlockSpec((256, _DH), lambda i: (i, 0)),
        out_shape=jax.ShapeDtypeStruct((_NP, _DH), jnp.float32),
    )(x_pad, w, degp, degp)


def _postA_body(p0, p1, hs, d0, d1, b, a_ref, s_ref, acc):
    i = pl.program_id(0)
    dinv = _dinv(d0[...], d1[...])
    z = dinv * (p0[...] + p1[...] + hs[...]) + b[0:1, :]
    a = _selu(z)
    a_ref[...] = a
    rows = i * 256 + lax.broadcasted_iota(jnp.int32, (256, 1), 0)
    am = jnp.where(rows < _N, a, 0.0)

    @pl.when(i == 0)
    def _():
        acc[...] = jnp.zeros_like(acc)

    acc[0:1, :] += jnp.sum(am, axis=0, keepdims=True)
    acc[1:2, :] += jnp.sum(am * am, axis=0, keepdims=True)

    @pl.when(i == _NB - 1)
    def _():
        s_ref[...] = acc[...]


def _tc_postA(parts, hs, degp, b):
    return pl.pallas_call(
        _postA_body,
        grid=(_NB,),
        in_specs=[
            pl.BlockSpec((256, _DH), lambda i: (i, 0)),
            pl.BlockSpec((256, _DH), lambda i: (_NB + i, 0)),
            pl.BlockSpec((256, _DH), lambda i: (i, 0)),
            pl.BlockSpec((256, _D), lambda i: (i, 0)),
            pl.BlockSpec((256, _D), lambda i: (_NB + i, 0)),
            pl.BlockSpec((1, _DH), lambda i: (0, 0)),
        ],
        out_specs=[
            pl.BlockSpec((256, _DH), lambda i: (i, 0)),
            pl.BlockSpec((8, _DH), lambda i: (0, 0)),
        ],
        out_shape=[
            jax.ShapeDtypeStruct((_NP, _DH), jnp.float32),
            jax.ShapeDtypeStruct((8, _DH), jnp.float32),
        ],
        scratch_shapes=[pltpu.VMEM((8, _DH), jnp.float32)],
    )(parts, parts, hs, degp, degp, b)


def _postB_body(a, s, g, be, w, d0, d1, o_ref):
    mu = s[0:1, :] * (1.0 / _N)
    var = s[1:2, :] * (1.0 / _N) - mu * mu
    rstd = lax.rsqrt(var + 1e-5)
    h = (a[...] - mu) * rstd * g[0:1, :] + be[0:1, :]
    dinv = _dinv(d0[...], d1[...])
    o_ref[...] = jnp.dot(h, w[...], precision=_P,
                         preferred_element_type=jnp.float32) * dinv


def _tc_postB(a, sums, g, be, w, degp):
    return pl.pallas_call(
        _postB_body,
        grid=(_NB,),
        in_specs=[
            pl.BlockSpec((256, _DH), lambda i: (i, 0)),
            pl.BlockSpec((8, _DH), lambda i: (0, 0)),
            pl.BlockSpec((1, _DH), lambda i: (0, 0)),
            pl.BlockSpec((1, _DH), lambda i: (0, 0)),
            pl.BlockSpec((_DH, _DH), lambda i: (0, 0)),
            pl.BlockSpec((256, _D), lambda i: (i, 0)),
            pl.BlockSpec((256, _D), lambda i: (_NB + i, 0)),
        ],
        out_specs=pl.BlockSpec((256, _DH), lambda i: (i, 0)),
        out_shape=jax.ShapeDtypeStruct((_NP, _DH), jnp.float32),
    )(a, sums, g, be, w, degp, degp)


def _final_body(p0, p1, hs, d0, d1, b, bt, mol, wa, wb, bf1, wf2, bf2,
                o_ref, hg_acc):
    i = pl.program_id(0)
    dinv = _dinv(d0[...], d1[...])
    z = dinv * (p0[...] + p1[...] + hs[...]) + b[0:1, :]
    a = _selu(z)
    oh = (bt[...] == lax.broadcasted_iota(jnp.int32, (256, _G), 1)
          ).astype(jnp.float32)
    part = lax.dot_general(oh, a, (((0,), (0,)), ((), ())), precision=_P,
                           preferred_element_type=jnp.float32)

    @pl.when(i == 0)
    def _():
        hg_acc[...] = jnp.zeros_like(hg_acc)

    hg_acc[...] += part

    @pl.when(i == _NB - 1)
    def _():
        hg = hg_acc[...]
        h = (jnp.dot(hg, wa[...], precision=_P,
                     preferred_element_type=jnp.float32)
             + jnp.dot(mol[...], wb[...], precision=_P,
                       preferred_element_type=jnp.float32)
             + bf1[0:1, :])
        h = _selu(h)
        res = jnp.dot(h, wf2[...], precision=_P,
                      preferred_element_type=jnp.float32)
        o_ref[...] = res[:, 0:1] + bf2[0, 0]


def _tc_final(parts, hs, degp, b, batch2d, mol, wa, wb, bf1, wf2p, bf2):
    return pl.pallas_call(
        _final_body,
        grid=(_NB,),
        in_specs=[
            pl.BlockSpec((256, _DH), lambda i: (i, 0)),
            pl.BlockSpec((256, _DH), lambda i: (_NB + i, 0)),
            pl.BlockSpec((256, _DH), lambda i: (i, 0)),
            pl.BlockSpec((256, _D), lambda i: (i, 0)),
            pl.BlockSpec((256, _D), lambda i: (_NB + i, 0)),
            pl.BlockSpec((1, _DH), lambda i: (0, 0)),
            pl.BlockSpec((256, 1), lambda i: (i, 0)),
            pl.BlockSpec((_G, _MF), lambda i: (0, 0)),
            pl.BlockSpec((_DH, _DH), lambda i: (0, 0)),
            pl.BlockSpec((_MF, _DH), lambda i: (0, 0)),
            pl.BlockSpec((1, _DH), lambda i: (0, 0)),
            pl.BlockSpec((_DH, _DH), lambda i: (0, 0)),
            pl.BlockSpec((1, 1), lambda i: (0, 0)),
        ],
        out_specs=pl.BlockSpec((_G, 1), lambda i: (0, 0)),
        out_shape=jax.ShapeDtypeStruct((_G, 1), jnp.float32),
        scratch_shapes=[pltpu.VMEM((_G, _DH), jnp.float32)],
    )(parts, parts, hs, degp, degp, b, batch2d, mol, wa, wb, bf1, wf2p, bf2)


def kernel(x, edge_index, batch, mol_feats, W1, b1, g1, be1, W2, b2, g2, be2,
           W3, b3, Wf1, bf1, Wf2, bf2):
    f32 = jnp.float32
    npad = _NP - _N
    epad = _EP - _E

    src = jnp.concatenate(
        [edge_index[0], jnp.full((epad,), _N, jnp.int32)])
    dst = jnp.concatenate(
        [edge_index[1], _N + (jnp.arange(epad, dtype=jnp.int32) % 128)])
    x_pad = jnp.concatenate([x, jnp.zeros((npad, _D), f32)], axis=0)
    batch2d = jnp.concatenate(
        [batch, jnp.full((npad,), _G, jnp.int32)]).reshape(_NP, 1)

    b1r = b1.reshape(1, _DH)
    b2r = b2.reshape(1, _DH)
    b3r = b3.reshape(1, _DH)
    g1r = g1.reshape(1, _DH)
    g2r = g2.reshape(1, _DH)
    be1r = be1.reshape(1, _DH)
    be2r = be2.reshape(1, _DH)
    bf1r = bf1.reshape(1, -1)
    wa = Wf1[:_DH]
    wb = Wf1[_DH:]
    wf2p = jnp.concatenate([Wf2, jnp.zeros((Wf2.shape[0], _DH - Wf2.shape[1]),
                                           f32)], axis=1)
    bf2r = bf2.reshape(1, 1)

    degp = _sc_degree(dst)

    hs1 = _tc_pre(x_pad, W1, degp)
    p1 = _sc_edge_scatter(hs1, src, dst)
    a1, s1 = _tc_postA(p1, hs1, degp, b1r)
    hs2 = _tc_postB(a1, s1, g1r, be1r, W2, degp)

    p2 = _sc_edge_scatter(hs2, src, dst)
    a2, s2 = _tc_postA(p2, hs2, degp, b2r)
    hs3 = _tc_postB(a2, s2, g2r, be2r, W3, degp)

    p3 = _sc_edge_scatter(hs3, src, dst)
    out = _tc_final(p3, hs3, degp, b3r, batch2d, mol_feats, wa, wb, bf1r,
                    wf2p, bf2r)
    return out
